# Initial kernel scaffold; baseline (speedup 1.0000x reference)
import jax
import jax.numpy as jnp
from jax.experimental import pallas as pl


def kernel(x, w_fixed, b_fixed, w_a, b_a, w_b, b_b):
    raise NotImplementedError("write your pallas kernel here")


# speedup vs baseline: 2.3520x; 2.3520x over previous
import functools

import jax
import jax.numpy as jnp
from jax.experimental import pallas as pl
from jax.experimental.pallas import tpu as pltpu


def _fused_kernel(x_ref, wc_ref, ssel_ref, wb_ref, u2_ref, bias_ref, ba_ref,
                  m_ref, am_ref, o_ref, *, W, Wa, HW, Ma):
    cout = wb_ref.shape[0]
    xv = x_ref[0]

    parts = []
    for t in range(9):
        kh, kw = divmod(t, 3)
        off = (kh - 1) * W + (kw - 1)
        r = pltpu.roll(xv, (-off) % HW, axis=1) if off != 0 else xv
        if t != 4:
            r = r * m_ref[t]
        parts.append(r)
    patches = jnp.concatenate(parts, axis=0)

    acc9 = jnp.dot(wc_ref[...], patches, preferred_element_type=jnp.float32)
    acc = acc9[:cout]
    v = acc9[cout:cout + 1]

    a_img = jnp.dot(v, ssel_ref[...],
                    preferred_element_type=jnp.float32) + ba_ref[...]
    aparts = []
    for t in range(9):
        kh, kw = divmod(t, 3)
        off = (kh - 1) * Wa + (kw - 1)
        r = pltpu.roll(a_img, (-off) % Ma, axis=1) if off != 0 else a_img
        if t != 4:
            r = r * am_ref[t]
        aparts.append(r)
    a9 = jnp.concatenate(aparts, axis=0)

    ls = jnp.dot(wb_ref[...], a9, preferred_element_type=jnp.float32)
    up = jnp.dot(ls, u2_ref[...], preferred_element_type=jnp.float32)

    o_ref[0] = (acc + up + bias_ref[...]).astype(o_ref.dtype)


def kernel(x, w_fixed, b_fixed, w_a, b_a, w_b, b_b):
    N, Cin, H, W = x.shape
    Cout = w_fixed.shape[0]
    HW = H * W
    Ha, Wa = H // 4, W // 4
    Ma = Ha * Wa
    dtype = x.dtype

    xf = x.reshape(N, Cin, HW)

    wc = jnp.concatenate([
        jnp.transpose(w_fixed, (0, 2, 3, 1)).reshape(Cout, Cin * 9),
        jnp.transpose(w_a, (0, 2, 3, 1)).reshape(1, Cin * 9),
    ], axis=0)
    wb9 = w_b.reshape(Cout, 9)
    bias = (b_fixed + b_b).reshape(Cout, 1)
    ba = b_a.reshape(1, 1)

    hh = jnp.arange(HW) // W
    ww = jnp.arange(HW) % W
    ha = jnp.arange(Ma) // Wa
    wa_ = jnp.arange(Ma) % Wa
    masks, amasks = [], []
    for t in range(9):
        kh, kw = divmod(t, 3)
        masks.append(((hh + kh - 1 >= 0) & (hh + kh - 1 < H)
                      & (ww + kw - 1 >= 0) & (ww + kw - 1 < W)))
        amasks.append(((ha + kh - 1 >= 0) & (ha + kh - 1 < Ha)
                       & (wa_ + kw - 1 >= 0) & (wa_ + kw - 1 < Wa)))
    m9 = jnp.stack(masks).reshape(9, 1, HW).astype(jnp.float32)
    am9 = jnp.stack(amasks).reshape(9, 1, Ma).astype(jnp.float32)

    q_of_m = (ha * 4) * W + wa_ * 4
    ssel = (jnp.arange(HW)[:, None] == q_of_m[None, :]).astype(jnp.float32)
    m_of_q = (hh // 4) * Wa + ww // 4
    u2 = (jnp.arange(Ma)[:, None] == m_of_q[None, :]).astype(jnp.float32)

    flops = int(N * (2 * (Cout + 1) * Cin * 9 * HW + 2 * HW * Ma
                     + 2 * Cout * 9 * Ma + 2 * Cout * Ma * HW))
    bytes_accessed = int(4 * (N * Cin * HW + N * Cout * HW + HW * Ma * 2
                              + 9 * HW + 9 * Ma))

    kern = functools.partial(_fused_kernel, W=W, Wa=Wa, HW=HW, Ma=Ma)
    out = pl.pallas_call(
        kern,
        out_shape=jax.ShapeDtypeStruct((N, Cout, HW), dtype),
        grid=(N,),
        in_specs=[
            pl.BlockSpec((1, Cin, HW), lambda n: (n, 0, 0)),
            pl.BlockSpec((Cout + 1, Cin * 9), lambda n: (0, 0)),
            pl.BlockSpec((HW, Ma), lambda n: (0, 0)),
            pl.BlockSpec((Cout, 9), lambda n: (0, 0)),
            pl.BlockSpec((Ma, HW), lambda n: (0, 0)),
            pl.BlockSpec((Cout, 1), lambda n: (0, 0)),
            pl.BlockSpec((1, 1), lambda n: (0, 0)),
            pl.BlockSpec((9, 1, HW), lambda n: (0, 0, 0)),
            pl.BlockSpec((9, 1, Ma), lambda n: (0, 0, 0)),
        ],
        out_specs=pl.BlockSpec((1, Cout, HW), lambda n: (n, 0, 0)),
        compiler_params=pltpu.CompilerParams(dimension_semantics=("parallel",)),
        cost_estimate=pl.CostEstimate(flops=flops, transcendentals=0,
                                      bytes_accessed=bytes_accessed),
    )(xf, wc, ssel, wb9, u2, bias, ba, m9, am9)

    return out.reshape(N, Cout, H, W)
